# baseline (speedup 1.0000x reference)
import jax
import jax.numpy as jnp
from jax import lax
from jax.experimental import pallas as pl
from jax.experimental.pallas import tpu as pltpu
from jax.experimental.pallas import tpu_sc as plsc

_NC = 2
_NS = 16
_L = 16
_NW = _NC * _NS
_B = 16384
_F = 32
_BPW = _B // _NW
_CHUNK = 128
_PH = 2
_PROWS = _BPW // _PH
_RB = 4


def _body(user_hbm, item_hbm, uemb_hbm, iemb_hbm, out_hbm,
          uidx_v, iidx_v, ublk_v, iblk_v, urows_v, irows_v, out_v, sem):
    wid = lax.axis_index("s") * _NC + lax.axis_index("c")
    base = wid * _BPW

    pltpu.sync_copy(user_hbm.at[pl.ds(base, _BPW)], uidx_v)
    pltpu.sync_copy(item_hbm.at[pl.ds(base, _BPW)], iidx_v)

    for j in range(_BPW // _CHUNK):
        for c in range(_CHUNK // _L):
            s = pl.ds(j * _CHUNK + c * _L, _L)
            d = pl.ds(c * _L, _L)
            ublk_v[j, d] = jnp.right_shift(uidx_v[s], 2)
            iblk_v[j, d] = jnp.right_shift(iidx_v[s], 2)

    lanes = lax.iota(jnp.int32, _L)

    for ph in range(_PH):
        copies = []
        for c in range(_PROWS // _CHUNK):
            j = ph * (_PROWS // _CHUNK) + c
            copies.append(pltpu.async_copy(
                uemb_hbm.at[ublk_v.at[j]],
                urows_v.at[pl.ds(c * _CHUNK, _CHUNK)], sem))
            copies.append(pltpu.async_copy(
                iemb_hbm.at[iblk_v.at[j]],
                irows_v.at[pl.ds(c * _CHUNK, _CHUNK)], sem))
        for cp in copies:
            cp.wait()

        def group(g, carry):
            r0 = g * _L
            uraw = uidx_v[pl.ds(ph * _PROWS + r0, _L)]
            iraw = iidx_v[pl.ds(ph * _PROWS + r0, _L)]
            ucol = jnp.left_shift(jnp.bitwise_and(uraw, _RB - 1), 5)
            icol = jnp.left_shift(jnp.bitwise_and(iraw, _RB - 1), 5)
            row = r0 + lanes
            acc = jnp.zeros((_L,), jnp.float32)
            for f in range(_F):
                u = plsc.load_gather(urows_v, [row, ucol + f])
                v = plsc.load_gather(irows_v, [row, icol + f])
                acc = acc + u * v
            out_v[pl.ds(ph * _PROWS + r0, _L)] = 1.0 / (1.0 + jnp.exp(-acc))
            return carry

        lax.fori_loop(0, _PROWS // _L, group, 0)

    pltpu.sync_copy(out_v, out_hbm.at[pl.ds(base, _BPW)])


def kernel(user, item, user_emb, item_emb):
    uemb2 = user_emb.reshape(user_emb.shape[0] // _RB, _RB * _F)
    iemb2 = item_emb.reshape(item_emb.shape[0] // _RB, _RB * _F)
    run = pl.kernel(
        _body,
        out_type=jax.ShapeDtypeStruct((_B,), jnp.float32),
        mesh=plsc.VectorSubcoreMesh(
            core_axis_name="c", subcore_axis_name="s",
            num_cores=_NC, num_subcores=_NS),
        scratch_types=[
            pltpu.VMEM((_BPW,), jnp.int32),
            pltpu.VMEM((_BPW,), jnp.int32),
            pltpu.VMEM((_BPW // _CHUNK, _CHUNK), jnp.int32),
            pltpu.VMEM((_BPW // _CHUNK, _CHUNK), jnp.int32),
            pltpu.VMEM((_PROWS, _RB * _F), jnp.float32),
            pltpu.VMEM((_PROWS, _RB * _F), jnp.float32),
            pltpu.VMEM((_BPW,), jnp.float32),
            pltpu.SemaphoreType.DMA,
        ],
        compiler_params=pltpu.CompilerParams(
            needs_layout_passes=False, use_tc_tiling_on_sc=False),
    )
    return run(user.astype(jnp.int32), item.astype(jnp.int32), uemb2, iemb2)


# speedup vs baseline: 1.0010x; 1.0010x over previous
import jax
import jax.numpy as jnp
from jax import lax
from jax.experimental import pallas as pl
from jax.experimental.pallas import tpu as pltpu
from jax.experimental.pallas import tpu_sc as plsc

_NC = 2
_NS = 16
_L = 16
_NW = _NC * _NS
_B = 16384
_F = 32
_BPW = _B // _NW
_CHUNK = 128
_PH = 2
_PROWS = _BPW // _PH
_RB = 4


def _body(user_hbm, item_hbm, uemb_hbm, iemb_hbm, out_hbm,
          uidx_v, iidx_v, ublk_v, iblk_v, urows_v, irows_v, out_v, sem):
    wid = lax.axis_index("s") * _NC + lax.axis_index("c")
    base = wid * _BPW

    pltpu.sync_copy(user_hbm.at[pl.ds(base, _BPW)], uidx_v)
    pltpu.sync_copy(item_hbm.at[pl.ds(base, _BPW)], iidx_v)

    for j in range(_BPW // _CHUNK):
        for c in range(_CHUNK // _L):
            s = pl.ds(j * _CHUNK + c * _L, _L)
            d = pl.ds(c * _L, _L)
            ublk_v[j, d] = jnp.right_shift(uidx_v[s], 2)
            iblk_v[j, d] = jnp.right_shift(iidx_v[s], 2)

    lanes = lax.iota(jnp.int32, _L)

    for ph in range(_PH):
        copies = []
        for c in range(_PROWS // _CHUNK):
            j = ph * (_PROWS // _CHUNK) + c
            copies.append(pltpu.async_copy(
                uemb_hbm.at[ublk_v.at[j]],
                urows_v.at[pl.ds(c * _CHUNK, _CHUNK)], sem))
            copies.append(pltpu.async_copy(
                iemb_hbm.at[iblk_v.at[j]],
                irows_v.at[pl.ds(c * _CHUNK, _CHUNK)], sem))
        for cp in copies:
            cp.wait()

        def group(g, carry):
            r0 = g * _L
            uraw = uidx_v[pl.ds(ph * _PROWS + r0, _L)]
            iraw = iidx_v[pl.ds(ph * _PROWS + r0, _L)]
            ucol = jnp.left_shift(jnp.bitwise_and(uraw, _RB - 1), 5)
            icol = jnp.left_shift(jnp.bitwise_and(iraw, _RB - 1), 5)
            row = r0 + lanes
            acc = jnp.zeros((_L,), jnp.float32)
            for f in range(_F):
                u = plsc.load_gather(urows_v, [row, ucol + f])
                v = plsc.load_gather(irows_v, [row, icol + f])
                acc = acc + u * v
            out_v[pl.ds(ph * _PROWS + r0, _L)] = 1.0 / (1.0 + jnp.exp(-acc))
            return carry

        lax.fori_loop(0, _PROWS // _L, group, 0)

    pltpu.sync_copy(out_v, out_hbm.at[pl.ds(base, _BPW)])


def kernel(user, item, user_emb, item_emb):
    uemb2 = user_emb.reshape(user_emb.shape[0] // _RB, _RB * _F)
    iemb2 = item_emb.reshape(item_emb.shape[0] // _RB, _RB * _F)
    run = pl.kernel(
        _body,
        out_type=jax.ShapeDtypeStruct((_B,), jnp.float32),
        mesh=plsc.VectorSubcoreMesh(
            core_axis_name="c", subcore_axis_name="s",
            num_cores=_NC, num_subcores=_NS),
        scratch_types=[
            pltpu.VMEM((_BPW,), jnp.int32),
            pltpu.VMEM((_BPW,), jnp.int32),
            pltpu.VMEM((_BPW // _CHUNK, _CHUNK), jnp.int32),
            pltpu.VMEM((_BPW // _CHUNK, _CHUNK), jnp.int32),
            pltpu.VMEM((_PROWS, _RB * _F), jnp.float32),
            pltpu.VMEM((_PROWS, _RB * _F), jnp.float32),
            pltpu.VMEM((_BPW,), jnp.float32),
            pltpu.SemaphoreType.DMA,
        ],
        compiler_params=pltpu.CompilerParams(
            needs_layout_passes=False, use_tc_tiling_on_sc=True),
    )
    return run(user.astype(jnp.int32), item.astype(jnp.int32), uemb2, iemb2)


# speedup vs baseline: 3.7622x; 3.7585x over previous
import jax
import jax.numpy as jnp
from jax import lax
from jax.experimental import pallas as pl
from jax.experimental.pallas import tpu as pltpu
from jax.experimental.pallas import tpu_sc as plsc

_NC = 2
_NS = 16
_L = 16
_NW = _NC * _NS
_B = 16384
_F = 32
_BPW = _B // _NW
_G = 4
_GR = 8
_TC = 128


def _fetch(emb_hbm, craw, tiles_v, sem):
    copies = []
    for r in range(_L):
        t = jnp.right_shift(craw[r], 7)
        for g in range(_G):
            copies.append(pltpu.async_copy(
                emb_hbm.at[g, :, pl.ds(t * _TC, _TC)],
                tiles_v.at[r * _G + g], sem))
    for cp in copies:
        cp.wait()


def _body(user_hbm, item_hbm, uemb_hbm, iemb_hbm, out_hbm,
          vidx_u, vidx_i, tiles_v, urow_v, out_v, sem):
    wid = lax.axis_index("s") * _NC + lax.axis_index("c")
    base = wid * _BPW
    lanes = lax.iota(jnp.int32, _L)

    pltpu.sync_copy(user_hbm.at[pl.ds(base, _BPW)], vidx_u)
    pltpu.sync_copy(item_hbm.at[pl.ds(base, _BPW)], vidx_i)

    def group(g0, carry):
        craw_u = vidx_u[pl.ds(g0 * _L, _L)]
        craw_i = vidx_i[pl.ds(g0 * _L, _L)]
        ccol_u = jnp.bitwise_and(craw_u, _TC - 1)
        ccol_i = jnp.bitwise_and(craw_i, _TC - 1)
        slot0 = lanes * _G

        _fetch(uemb_hbm, craw_u, tiles_v, sem)
        for f in range(_F):
            w = plsc.load_gather(
                tiles_v, [slot0 + f // _GR, jnp.full((_L,), f % _GR, jnp.int32),
                          ccol_u])
            urow_v[pl.ds(f * _L, _L)] = w

        _fetch(iemb_hbm, craw_i, tiles_v, sem)
        acc = jnp.zeros((_L,), jnp.float32)
        for f in range(_F):
            w = plsc.load_gather(
                tiles_v, [slot0 + f // _GR, jnp.full((_L,), f % _GR, jnp.int32),
                          ccol_i])
            acc = acc + w * urow_v[pl.ds(f * _L, _L)]
        out_v[pl.ds(g0 * _L, _L)] = 1.0 / (1.0 + jnp.exp(-acc))
        return carry

    lax.fori_loop(0, _BPW // _L, group, 0)

    pltpu.sync_copy(out_v, out_hbm.at[pl.ds(base, _BPW)])


def kernel(user, item, user_emb, item_emb):
    uemb3 = user_emb.T.reshape(_G, _GR, user_emb.shape[0])
    iemb3 = item_emb.T.reshape(_G, _GR, item_emb.shape[0])
    run = pl.kernel(
        _body,
        out_type=jax.ShapeDtypeStruct((_B,), jnp.float32),
        mesh=plsc.VectorSubcoreMesh(
            core_axis_name="c", subcore_axis_name="s",
            num_cores=_NC, num_subcores=_NS),
        scratch_types=[
            pltpu.VMEM((_BPW,), jnp.int32),
            pltpu.VMEM((_BPW,), jnp.int32),
            pltpu.VMEM((_L * _G, _GR, _TC), jnp.float32),
            pltpu.VMEM((_F * _L,), jnp.float32),
            pltpu.VMEM((_BPW,), jnp.float32),
            pltpu.SemaphoreType.DMA,
        ],
        compiler_params=pltpu.CompilerParams(
            needs_layout_passes=False, use_tc_tiling_on_sc=True),
    )
    return run(user.astype(jnp.int32), item.astype(jnp.int32), uemb3, iemb3)
